# baseline (speedup 1.0000x reference)
import functools

import jax
import jax.numpy as jnp
from jax import lax
from jax.experimental import pallas as pl
from jax.experimental.pallas import tpu as pltpu
from jax.experimental.pallas import tpu_sc as plsc

NC = 2
NS = 16
NW = NC * NS
LANES = 16

BATCH = 16384
OUT_DIM = 32
B_PER_W = BATCH // NW
CHUNK = 128
N_CHUNKS = B_PER_W // CHUNK


def _sc_kernel(i_hbm, j_hbm, u_hbm, v_hbm, out_hbm,
               idx_u, idx_v, u_rows, v_rows, out_v, sem):
    wid = lax.axis_index("s") * NC + lax.axis_index("c")
    base = wid * B_PER_W

    pltpu.sync_copy(i_hbm.at[wid], idx_u)
    pltpu.sync_copy(j_hbm.at[wid], idx_v)

    copies = []
    for k in range(N_CHUNKS):
        copies.append(pltpu.async_copy(
            u_hbm.at[idx_u.at[k]], u_rows.at[pl.ds(k * CHUNK, CHUNK)], sem))
        copies.append(pltpu.async_copy(
            v_hbm.at[idx_v.at[k]], v_rows.at[pl.ds(k * CHUNK, CHUNK)], sem))
    for c in copies:
        c.wait()

    def group_body(g, _):
        rowi = g * LANES + lax.iota(jnp.int32, LANES)
        acc = jnp.zeros((LANES,), jnp.float32)
        for d in range(OUT_DIM):
            cols = jnp.full((LANES,), d, jnp.int32)
            ud = plsc.load_gather(u_rows, [rowi, cols])
            vd = plsc.load_gather(v_rows, [rowi, cols])
            acc = acc + ud * vd
        out_v[pl.ds(g * LANES, LANES)] = acc
        return 0

    lax.fori_loop(0, B_PER_W // LANES, group_body, 0)

    pltpu.sync_copy(out_v, out_hbm.at[pl.ds(base, B_PER_W)])


@jax.jit
def _run(i3, j3, u_emb, v_emb):
    mesh = plsc.VectorSubcoreMesh(
        core_axis_name="c", subcore_axis_name="s",
        num_cores=NC, num_subcores=NS)
    f = pl.kernel(
        _sc_kernel,
        out_type=jax.ShapeDtypeStruct((BATCH,), jnp.float32),
        mesh=mesh,
        compiler_params=pltpu.CompilerParams(
            needs_layout_passes=False, use_tc_tiling_on_sc=False),
        scratch_types=[
            pltpu.VMEM((N_CHUNKS, CHUNK), jnp.int32),
            pltpu.VMEM((N_CHUNKS, CHUNK), jnp.int32),
            pltpu.VMEM((B_PER_W, OUT_DIM), jnp.float32),
            pltpu.VMEM((B_PER_W, OUT_DIM), jnp.float32),
            pltpu.VMEM((B_PER_W,), jnp.float32),
            pltpu.SemaphoreType.DMA,
        ],
    )
    return f(i3, j3, u_emb, v_emb)


def kernel(i, j, u_emb, v_emb):
    i3 = i.astype(jnp.int32).reshape(NW, N_CHUNKS, CHUNK)
    j3 = j.astype(jnp.int32).reshape(NW, N_CHUNKS, CHUNK)
    return _run(i3, j3, u_emb, v_emb)
